# initial kernel scaffold (unmeasured)
import jax
import jax.numpy as jnp
from jax import lax
from jax.experimental import pallas as pl
from jax.experimental.pallas import tpu as pltpu

N_DEV = 8
B_LOC = 2
SQ = 256
SKV = 256
HQ = 32
DH = 64
D_MODEL = 512
G = HQ // N_DEV
GD = G * DH
D_HID = N_DEV * GD


def kernel(x, Wq, K_ext, V_ext, Wo):
    def body(x_ref, wq_ref, k_hbm, v_hbm, wo_ref, out_ref,
             wq_g, wo_g, q_s, ctx_s, k_s, v_s,
             wq_send, wq_recv, wo_send, wo_recv, k_sems, v_sems):
        my = lax.axis_index("i")
        right = lax.rem(my + 1, N_DEV)
        left = lax.rem(my + N_DEV - 1, N_DEV)

        def kv_copies(b):
            bg = my * B_LOC + b
            ops = []
            for h in range(HQ):
                ops.append(pltpu.make_async_copy(
                    k_hbm.at[bg, :, h, :], k_s.at[b, h], k_sems.at[b, h]))
                ops.append(pltpu.make_async_copy(
                    v_hbm.at[bg, :, h, :], v_s.at[b, h], v_sems.at[b, h]))
            return ops

        for b in range(B_LOC):
            for op in kv_copies(b):
                op.start()

        pl.store(wq_g, (pl.ds(my, 1), slice(None), slice(None)),
                 wq_ref[...].astype(jnp.bfloat16)[None])
        pl.store(wo_g, (pl.ds(my, 1), slice(None), slice(None)),
                 wo_ref[...].astype(jnp.bfloat16)[None])

        barrier_sem = pltpu.get_barrier_semaphore()
        for nbr in (left, right):
            pl.semaphore_signal(barrier_sem, inc=1, device_id=(nbr,),
                                device_id_type=pl.DeviceIdType.MESH)
        pl.semaphore_wait(barrier_sem, 2)

        for h in range(N_DEV - 1):
            src = lax.rem(my + N_DEV - h, N_DEV)
            rdmas = []
            for gref, ssem, rsem in ((wq_g, wq_send, wq_recv),
                                     (wo_g, wo_send, wo_recv)):
                rdma = pltpu.make_async_remote_copy(
                    src_ref=gref.at[src],
                    dst_ref=gref.at[src],
                    send_sem=ssem.at[h],
                    recv_sem=rsem.at[h],
                    device_id=(right,),
                    device_id_type=pl.DeviceIdType.MESH,
                )
                rdma.start()
                rdmas.append(rdma)
            for rdma in rdmas:
                rdma.wait()

        qb = lax.broadcasted_iota(jnp.int32, (SQ, SKV), 0) // 64
        kb = lax.broadcasted_iota(jnp.int32, (SQ, SKV), 1) // 64
        mask = kb <= qb

        for b in range(B_LOC):
            xb = x_ref[b].astype(jnp.bfloat16)
            for j in range(N_DEV):
                qj = jnp.dot(xb, wq_g[j], preferred_element_type=jnp.float32)
                q_s[:, j * GD:(j + 1) * GD] = qj.astype(jnp.bfloat16)
            for op in kv_copies(b):
                op.wait()

            def head_body(hh, carry):
                q = pl.load(q_s, (slice(None), pl.ds(hh * DH, DH)))
                k = pl.load(k_s, (b, pl.ds(hh, 1), slice(None), slice(None)))
                k = k.reshape(SKV, DH).astype(jnp.bfloat16)
                s = lax.dot_general(
                    q, k, (((1,), (1,)), ((), ())),
                    preferred_element_type=jnp.float32) * 0.125
                s = jnp.where(mask, s, -1e9)
                m = jnp.max(s, axis=1, keepdims=True)
                e = jnp.exp(s - m)
                w = (e / jnp.sum(e, axis=1, keepdims=True)).astype(jnp.bfloat16)
                v = pl.load(v_s, (b, pl.ds(hh, 1), slice(None), slice(None)))
                v = v.reshape(SKV, DH).astype(jnp.bfloat16)
                ctx = jnp.dot(w, v, preferred_element_type=jnp.float32)
                pl.store(ctx_s, (slice(None), pl.ds(hh * DH, DH)),
                         ctx.astype(jnp.bfloat16))
                return carry

            lax.fori_loop(0, HQ, head_body, 0)

            acc = jnp.zeros((SQ, D_MODEL), jnp.float32)
            for j in range(N_DEV):
                acc = acc + jnp.dot(ctx_s[:, j * GD:(j + 1) * GD], wo_g[j],
                                    preferred_element_type=jnp.float32)
            out_ref[b] = acc

    return pl.pallas_call(
        body,
        out_shape=jax.ShapeDtypeStruct((B_LOC, SQ, D_MODEL), jnp.float32),
        in_specs=[
            pl.BlockSpec(memory_space=pltpu.VMEM),
            pl.BlockSpec(memory_space=pltpu.VMEM),
            pl.BlockSpec(memory_space=pltpu.ANY),
            pl.BlockSpec(memory_space=pltpu.ANY),
            pl.BlockSpec(memory_space=pltpu.VMEM),
        ],
        out_specs=pl.BlockSpec(memory_space=pltpu.VMEM),
        scratch_shapes=[
            pltpu.VMEM((N_DEV, D_MODEL, GD), jnp.bfloat16),
            pltpu.VMEM((N_DEV, GD, D_MODEL), jnp.bfloat16),
            pltpu.VMEM((SQ, D_HID), jnp.bfloat16),
            pltpu.VMEM((SQ, D_HID), jnp.bfloat16),
            pltpu.VMEM((B_LOC, HQ, SKV, DH), jnp.float32),
            pltpu.VMEM((B_LOC, HQ, SKV, DH), jnp.float32),
            pltpu.SemaphoreType.DMA((N_DEV - 1,)),
            pltpu.SemaphoreType.DMA((N_DEV - 1,)),
            pltpu.SemaphoreType.DMA((N_DEV - 1,)),
            pltpu.SemaphoreType.DMA((N_DEV - 1,)),
            pltpu.SemaphoreType.DMA((B_LOC, HQ)),
            pltpu.SemaphoreType.DMA((B_LOC, HQ)),
        ],
        compiler_params=pltpu.CompilerParams(collective_id=0),
    )(x, Wq, K_ext, V_ext, Wo)


# baseline (device time: 216542 ns/iter reference)
import jax
import jax.numpy as jnp
from jax import lax
from jax.experimental import pallas as pl
from jax.experimental.pallas import tpu as pltpu

N_DEV = 8
B_LOC = 2
SQ = 256
SKV = 256
HQ = 32
DH = 64
D_MODEL = 512
G = HQ // N_DEV
GD = G * DH
D_HID = N_DEV * GD


def kernel(x, Wq, K_ext, V_ext, Wo):
    def body(x_ref, wq_ref, k_hbm, v_hbm, wo_ref, out_ref,
             wq_g, wo_g, q_s, ctx_s, k_s, v_s,
             wq_send, wq_recv, wo_send, wo_recv, k_sems, v_sems):
        my = lax.axis_index("i")
        right = lax.rem(my + 1, N_DEV)
        left = lax.rem(my + N_DEV - 1, N_DEV)

        def kv_copies(b):
            bg = my * B_LOC + b
            ops = []
            for h in range(HQ):
                ops.append(pltpu.make_async_copy(
                    k_hbm.at[bg, :, h, :], k_s.at[b, h], k_sems.at[b, h]))
                ops.append(pltpu.make_async_copy(
                    v_hbm.at[bg, :, h, :], v_s.at[b, h], v_sems.at[b, h]))
            return ops

        for b in range(B_LOC):
            for op in kv_copies(b):
                op.start()

        wq_g[pl.ds(my, 1), :, :] = wq_ref[...].astype(jnp.bfloat16)[None]
        wo_g[pl.ds(my, 1), :, :] = wo_ref[...].astype(jnp.bfloat16)[None]

        barrier_sem = pltpu.get_barrier_semaphore()
        for nbr in (left, right):
            pl.semaphore_signal(barrier_sem, inc=1, device_id=(nbr,),
                                device_id_type=pl.DeviceIdType.MESH)
        pl.semaphore_wait(barrier_sem, 2)

        for h in range(N_DEV - 1):
            src = lax.rem(my + N_DEV - h, N_DEV)
            rdmas = []
            for gref, ssem, rsem in ((wq_g, wq_send, wq_recv),
                                     (wo_g, wo_send, wo_recv)):
                rdma = pltpu.make_async_remote_copy(
                    src_ref=gref.at[src],
                    dst_ref=gref.at[src],
                    send_sem=ssem.at[h],
                    recv_sem=rsem.at[h],
                    device_id=(right,),
                    device_id_type=pl.DeviceIdType.MESH,
                )
                rdma.start()
                rdmas.append(rdma)
            for rdma in rdmas:
                rdma.wait()

        qb = lax.broadcasted_iota(jnp.int32, (SQ, SKV), 0) // 64
        kb = lax.broadcasted_iota(jnp.int32, (SQ, SKV), 1) // 64
        mask = kb <= qb

        for b in range(B_LOC):
            xb = x_ref[b].astype(jnp.bfloat16)
            for j in range(N_DEV):
                qj = jnp.dot(xb, wq_g[j], preferred_element_type=jnp.float32)
                qj = qj.astype(jnp.bfloat16)
                for hh in range(G):
                    q_s[j * G + hh] = qj[:, hh * DH:(hh + 1) * DH]
            for op in kv_copies(b):
                op.wait()

            def head_body(hh, carry):
                q = q_s[pl.ds(hh, 1), :, :].reshape(SQ, DH)
                k = k_s[b, pl.ds(hh, 1), :, :]
                k = k.reshape(SKV, DH).astype(jnp.bfloat16)
                s = lax.dot_general(
                    q, k, (((1,), (1,)), ((), ())),
                    preferred_element_type=jnp.float32) * 0.125
                s = jnp.where(mask, s, -1e9)
                m = jnp.max(s, axis=1, keepdims=True)
                e = jnp.exp(s - m)
                w = (e / jnp.sum(e, axis=1, keepdims=True)).astype(jnp.bfloat16)
                v = v_s[b, pl.ds(hh, 1), :, :]
                v = v.reshape(SKV, DH).astype(jnp.bfloat16)
                ctx = jnp.dot(w, v, preferred_element_type=jnp.float32)
                ctx_s[pl.ds(hh, 1), :, :] = ctx.astype(jnp.bfloat16)[None]
                return carry

            lax.fori_loop(0, HQ, head_body, 0)

            acc = jnp.zeros((SQ, D_MODEL), jnp.float32)
            for j in range(N_DEV):
                ctxj = jnp.concatenate(
                    [ctx_s[j * G + hh] for hh in range(G)], axis=1)
                acc = acc + jnp.dot(ctxj, wo_g[j],
                                    preferred_element_type=jnp.float32)
            out_ref[b] = acc

    return pl.pallas_call(
        body,
        out_shape=jax.ShapeDtypeStruct((B_LOC, SQ, D_MODEL), jnp.float32),
        in_specs=[
            pl.BlockSpec(memory_space=pltpu.VMEM),
            pl.BlockSpec(memory_space=pltpu.VMEM),
            pl.BlockSpec(memory_space=pl.ANY),
            pl.BlockSpec(memory_space=pl.ANY),
            pl.BlockSpec(memory_space=pltpu.VMEM),
        ],
        out_specs=pl.BlockSpec(memory_space=pltpu.VMEM),
        scratch_shapes=[
            pltpu.VMEM((N_DEV, D_MODEL, GD), jnp.bfloat16),
            pltpu.VMEM((N_DEV, GD, D_MODEL), jnp.bfloat16),
            pltpu.VMEM((HQ, SQ, DH), jnp.bfloat16),
            pltpu.VMEM((HQ, SQ, DH), jnp.bfloat16),
            pltpu.VMEM((B_LOC, HQ, SKV, DH), jnp.float32),
            pltpu.VMEM((B_LOC, HQ, SKV, DH), jnp.float32),
            pltpu.SemaphoreType.DMA((N_DEV - 1,)),
            pltpu.SemaphoreType.DMA((N_DEV - 1,)),
            pltpu.SemaphoreType.DMA((N_DEV - 1,)),
            pltpu.SemaphoreType.DMA((N_DEV - 1,)),
            pltpu.SemaphoreType.DMA((B_LOC, HQ)),
            pltpu.SemaphoreType.DMA((B_LOC, HQ)),
        ],
        compiler_params=pltpu.CompilerParams(collective_id=0),
    )(x, Wq, K_ext, V_ext, Wo)


# device time: 163568 ns/iter; 1.3239x vs baseline; 1.3239x over previous
import os

import jax
import jax.numpy as jnp
from jax import lax
from jax.experimental import pallas as pl
from jax.experimental.pallas import tpu as pltpu

_DISABLE_RING = os.environ.get("KERNEL_DISABLE_RING") == "1"
_DISABLE_COMPUTE = os.environ.get("KERNEL_DISABLE_COMPUTE") == "1"
_DISABLE_KV = os.environ.get("KERNEL_DISABLE_KV") == "1"

N_DEV = 8
B_LOC = 2
SQ = 256
SKV = 256
HQ = 32
DH = 64
D_MODEL = 512
G = HQ // N_DEV
GD = G * DH
D_HID = N_DEV * GD


def kernel(x, Wq, K_ext, V_ext, Wo):
    def body(x_ref, wq_ref, k_hbm, v_hbm, wo_ref, out_ref,
             wq_g, wo_g, q_s, ctx_s, k_s, v_s,
             wq_send, wq_recv, wo_send, wo_recv, k_sems, v_sems):
        my = lax.axis_index("i")
        right = lax.rem(my + 1, N_DEV)
        left = lax.rem(my + N_DEV - 1, N_DEV)

        def kv_copies(b):
            bg = my * B_LOC + b
            ops = []
            for h in range(HQ):
                ops.append(pltpu.make_async_copy(
                    k_hbm.at[bg, :, h, :], k_s.at[b, h], k_sems.at[b, h]))
                ops.append(pltpu.make_async_copy(
                    v_hbm.at[bg, :, h, :], v_s.at[b, h], v_sems.at[b, h]))
            return ops

        if not _DISABLE_KV:
            for b in range(B_LOC):
                for op in kv_copies(b):
                    op.start()

        wq_g[pl.ds(my, 1), :, :] = wq_ref[...].astype(jnp.bfloat16)[None]
        wo_g[pl.ds(my, 1), :, :] = wo_ref[...].astype(jnp.bfloat16)[None]

        if _DISABLE_RING:
            ring_hops = 0
        else:
            ring_hops = N_DEV - 1

        barrier_sem = pltpu.get_barrier_semaphore()
        for nbr in (left, right):
            pl.semaphore_signal(barrier_sem, inc=1, device_id=(nbr,),
                                device_id_type=pl.DeviceIdType.MESH)
        pl.semaphore_wait(barrier_sem, 2)

        for h in range(ring_hops):
            src = lax.rem(my + N_DEV - h, N_DEV)
            rdmas = []
            for gref, ssem, rsem in ((wq_g, wq_send, wq_recv),
                                     (wo_g, wo_send, wo_recv)):
                rdma = pltpu.make_async_remote_copy(
                    src_ref=gref.at[src],
                    dst_ref=gref.at[src],
                    send_sem=ssem.at[h],
                    recv_sem=rsem.at[h],
                    device_id=(right,),
                    device_id_type=pl.DeviceIdType.MESH,
                )
                rdma.start()
                rdmas.append(rdma)
            for rdma in rdmas:
                rdma.wait()

        qb = lax.broadcasted_iota(jnp.int32, (SQ, SKV), 0) // 64
        kb = lax.broadcasted_iota(jnp.int32, (SQ, SKV), 1) // 64
        mask = kb <= qb

        for b in range(B_LOC) if not _DISABLE_COMPUTE else ():
            xb = x_ref[b].astype(jnp.bfloat16)
            for j in range(N_DEV):
                qj = jnp.dot(xb, wq_g[j], preferred_element_type=jnp.float32)
                qj = qj.astype(jnp.bfloat16)
                for hh in range(G):
                    q_s[j * G + hh] = qj[:, hh * DH:(hh + 1) * DH]
            if not _DISABLE_KV:
                for op in kv_copies(b):
                    op.wait()

            def head_body(hh, carry):
                q = q_s[pl.ds(hh, 1), :, :].reshape(SQ, DH)
                k = k_s[b, pl.ds(hh, 1), :, :]
                k = k.reshape(SKV, DH).astype(jnp.bfloat16)
                s = lax.dot_general(
                    q, k, (((1,), (1,)), ((), ())),
                    preferred_element_type=jnp.float32) * 0.125
                s = jnp.where(mask, s, -1e9)
                m = jnp.max(s, axis=1, keepdims=True)
                e = jnp.exp(s - m)
                w = (e / jnp.sum(e, axis=1, keepdims=True)).astype(jnp.bfloat16)
                v = v_s[b, pl.ds(hh, 1), :, :]
                v = v.reshape(SKV, DH).astype(jnp.bfloat16)
                ctx = jnp.dot(w, v, preferred_element_type=jnp.float32)
                ctx_s[pl.ds(hh, 1), :, :] = ctx.astype(jnp.bfloat16)[None]
                return carry

            lax.fori_loop(0, HQ, head_body, 0)

            acc = jnp.zeros((SQ, D_MODEL), jnp.float32)
            for j in range(N_DEV):
                ctxj = jnp.concatenate(
                    [ctx_s[j * G + hh] for hh in range(G)], axis=1)
                acc = acc + jnp.dot(ctxj, wo_g[j],
                                    preferred_element_type=jnp.float32)
            out_ref[b] = acc

        if _DISABLE_COMPUTE:
            out_ref[...] = jnp.zeros((B_LOC, SQ, D_MODEL), jnp.float32)

    return pl.pallas_call(
        body,
        out_shape=jax.ShapeDtypeStruct((B_LOC, SQ, D_MODEL), jnp.float32),
        in_specs=[
            pl.BlockSpec(memory_space=pltpu.VMEM),
            pl.BlockSpec(memory_space=pltpu.VMEM),
            pl.BlockSpec(memory_space=pl.ANY),
            pl.BlockSpec(memory_space=pl.ANY),
            pl.BlockSpec(memory_space=pltpu.VMEM),
        ],
        out_specs=pl.BlockSpec(memory_space=pltpu.VMEM),
        scratch_shapes=[
            pltpu.VMEM((N_DEV, D_MODEL, GD), jnp.bfloat16),
            pltpu.VMEM((N_DEV, GD, D_MODEL), jnp.bfloat16),
            pltpu.VMEM((HQ, SQ, DH), jnp.bfloat16),
            pltpu.VMEM((HQ, SQ, DH), jnp.bfloat16),
            pltpu.VMEM((B_LOC, HQ, SKV, DH), jnp.float32),
            pltpu.VMEM((B_LOC, HQ, SKV, DH), jnp.float32),
            pltpu.SemaphoreType.DMA((N_DEV - 1,)),
            pltpu.SemaphoreType.DMA((N_DEV - 1,)),
            pltpu.SemaphoreType.DMA((N_DEV - 1,)),
            pltpu.SemaphoreType.DMA((N_DEV - 1,)),
            pltpu.SemaphoreType.DMA((B_LOC, HQ)),
            pltpu.SemaphoreType.DMA((B_LOC, HQ)),
        ],
        compiler_params=pltpu.CompilerParams(collective_id=0),
    )(x, Wq, K_ext, V_ext, Wo)
